# baseline (speedup 1.0000x reference)
import functools

import jax
import jax.numpy as jnp
from jax import lax
from jax.experimental import pallas as pl
from jax.experimental.pallas import tpu as pltpu
from jax.experimental.pallas import tpu_sc as plsc

_N, _E, _D, _G = 10000, 320000, 128, 256
_H = _D // 2
_MLP_H = 128
_NPAD = 10240
_CW = 128
_NGRP = 2
_GCH = 81
_NCHUNK = _NGRP * _GCH
_EPAD = 16 * _NCHUNK * _CW
_ROWS_PER_TILE = _NPAD // 16
_SROWS = 128
_NBUF = 3

_MESH = plsc.VectorSubcoreMesh(core_axis_name="c", subcore_axis_name="s")


@functools.partial(
    pl.kernel,
    out_type=jax.ShapeDtypeStruct((2, _NPAD, _H), jnp.float32),
    mesh=_MESH,
    compiler_params=pltpu.CompilerParams(use_tc_tiling_on_sc=False),
    scratch_types=[
        pltpu.VMEM_SHARED((_NPAD, _H), jnp.float32),
        pltpu.VMEM_SHARED((_NPAD, _H), jnp.float32),
        pltpu.VMEM((_GCH, _CW), jnp.int32),
        pltpu.VMEM((_GCH, _CW), jnp.int32),
        pltpu.VMEM((_NBUF, _CW, _H), jnp.float32),
        pltpu.SemaphoreType.DMA((_NBUF,)),
        pltpu.SemaphoreType.DMA((_NBUF,)),
    ],
)
def _sc_aggregate(h_hbm, src_hbm, dst_hbm, zrows_hbm, out_hbm,
                  agg_sh, h_sh, src_v, dst_v, rows_v, sem_g, sem_s):
    c = lax.axis_index("c")
    s = lax.axis_index("s")
    npieces = _ROWS_PER_TILE // _SROWS

    pltpu.sync_copy(zrows_hbm, rows_v.at[0])

    def zstep(k, carry):
        pltpu.sync_copy(
            rows_v.at[0],
            agg_sh.at[pl.ds(s * _ROWS_PER_TILE + k * _SROWS, _SROWS)])
        return carry

    lax.fori_loop(0, npieces, zstep, 0)

    pltpu.async_copy(h_hbm.at[c, pl.ds(s * _ROWS_PER_TILE, _SROWS)],
                     rows_v.at[1], sem_g.at[1])
    for k in range(npieces):
        b = 1 + (k % 2)
        bn = 1 + ((k + 1) % 2)
        base = s * _ROWS_PER_TILE + k * _SROWS
        pltpu.make_async_copy(h_hbm.at[c, pl.ds(base, _SROWS)], rows_v.at[b],
                              sem_g.at[b]).wait()
        if k + 1 < npieces:
            pltpu.async_copy(
                h_hbm.at[c, pl.ds(base + _SROWS, _SROWS)], rows_v.at[bn],
                sem_g.at[bn])
        pltpu.sync_copy(rows_v.at[b], h_sh.at[pl.ds(base, _SROWS)])
    plsc.subcore_barrier()

    for g in range(_NGRP):
        pltpu.sync_copy(src_hbm.at[s, pl.ds(g * _GCH, _GCH)], src_v)
        pltpu.sync_copy(dst_hbm.at[s, pl.ds(g * _GCH, _GCH)], dst_v)

        for b in range(_NBUF):
            pltpu.async_copy(h_sh.at[src_v.at[b]], rows_v.at[b], sem_g.at[b])

        def outer(o, carry):
            for b in range(_NBUF):
                j = o * _NBUF + b
                pltpu.make_async_copy(h_sh.at[src_v.at[j]], rows_v.at[b],
                                      sem_g.at[b]).wait()
                pltpu.async_copy(rows_v.at[b], agg_sh.at[dst_v.at[j]],
                                 sem_s.at[b], add=True)
                bn = (b + 1) % _NBUF

                @pl.when((j >= 2) & (j <= _GCH - 2))
                def _():
                    pltpu.make_async_copy(rows_v.at[bn],
                                          agg_sh.at[dst_v.at[j - 2]],
                                          sem_s.at[bn]).wait()
                    from_hbm = (((j + 1) // _NBUF) % 4) == 3

                    @pl.when(from_hbm)
                    def _():
                        pltpu.async_copy(h_hbm.at[c].at[src_v.at[j + 1]],
                                         rows_v.at[bn], sem_g.at[bn])

                    @pl.when(jnp.logical_not(from_hbm))
                    def _():
                        pltpu.async_copy(h_sh.at[src_v.at[j + 1]],
                                         rows_v.at[bn], sem_g.at[bn])
            return carry

        lax.fori_loop(0, _GCH // _NBUF, outer, 0)
        for i in range(_NBUF):
            pltpu.make_async_copy(rows_v.at[i],
                                  agg_sh.at[dst_v.at[_GCH - _NBUF + i]],
                                  sem_s.at[i]).wait()

    plsc.subcore_barrier()

    pltpu.sync_copy(agg_sh.at[pl.ds(s * _ROWS_PER_TILE, _SROWS)], rows_v.at[0])
    for k in range(npieces):
        b = k % 2
        bn = (k + 1) % 2
        base = s * _ROWS_PER_TILE + k * _SROWS
        if k + 1 < npieces:
            pltpu.sync_copy(agg_sh.at[pl.ds(base + _SROWS, _SROWS)],
                            rows_v.at[bn])
        pltpu.sync_copy(rows_v.at[b], out_hbm.at[c, pl.ds(base, _SROWS)])


_BR = 1024


def _tc_layer_body(a_ref, h_ref, w_ref, b_ref, r_ref, rb_ref, o_ref):
    a = jnp.concatenate([a_ref[0], a_ref[1]], axis=1)
    hf = jnp.concatenate([h_ref[0], h_ref[1]], axis=1)
    conv = jnp.dot(a, w_ref[...], preferred_element_type=jnp.float32) + b_ref[...]
    res = jnp.dot(hf, r_ref[...], preferred_element_type=jnp.float32) + rb_ref[...]
    hn = jnp.maximum(conv, 0.0) + jnp.maximum(res, 0.0)
    o_ref[0] = hn[:, :_H]
    o_ref[1] = hn[:, _H:]


_tc_layer = pl.pallas_call(
    _tc_layer_body,
    grid=(_NPAD // _BR,),
    in_specs=[
        pl.BlockSpec((2, _BR, _H), lambda i: (0, i, 0)),
        pl.BlockSpec((2, _BR, _H), lambda i: (0, i, 0)),
        pl.BlockSpec((_D, _D), lambda i: (0, 0)),
        pl.BlockSpec((1, _D), lambda i: (0, 0)),
        pl.BlockSpec((_D, _D), lambda i: (0, 0)),
        pl.BlockSpec((1, _D), lambda i: (0, 0)),
    ],
    out_specs=pl.BlockSpec((2, _BR, _H), lambda i: (0, i, 0)),
    out_shape=jax.ShapeDtypeStruct((2, _NPAD, _H), jnp.float32),
)


def _tc_final_body(a_ref, h_ref, w_ref, b_ref, r_ref, rb_ref, n2g_ref,
                   wm1_ref, bm1_ref, wm2_ref, bm2_ref, o_ref, acc_ref):
    i = pl.program_id(0)
    a = jnp.concatenate([a_ref[0], a_ref[1]], axis=1)
    hf = jnp.concatenate([h_ref[0], h_ref[1]], axis=1)
    conv = jnp.dot(a, w_ref[...], preferred_element_type=jnp.float32) + b_ref[...]
    res = jnp.dot(hf, r_ref[...], preferred_element_type=jnp.float32) + rb_ref[...]
    hn = jnp.maximum(conv, 0.0) + jnp.maximum(res, 0.0)
    gid = lax.broadcasted_iota(jnp.int32, (_G, _BR), 0)
    onehot = (gid == n2g_ref[0]).astype(jnp.float32)
    part = jnp.dot(onehot, hn, preferred_element_type=jnp.float32)

    @pl.when(i == 0)
    def _():
        acc_ref[...] = jnp.zeros_like(acc_ref)

    acc_ref[...] += part

    @pl.when(i == _NPAD // _BR - 1)
    def _():
        mid = jnp.maximum(
            jnp.dot(acc_ref[...], wm1_ref[...],
                    preferred_element_type=jnp.float32) + bm1_ref[...], 0.0)
        o_ref[...] = jnp.dot(mid, wm2_ref[...],
                             preferred_element_type=jnp.float32) + bm2_ref[...]


_tc_final = pl.pallas_call(
    _tc_final_body,
    grid=(_NPAD // _BR,),
    in_specs=[
        pl.BlockSpec((2, _BR, _H), lambda i: (0, i, 0)),
        pl.BlockSpec((2, _BR, _H), lambda i: (0, i, 0)),
        pl.BlockSpec((_D, _D), lambda i: (0, 0)),
        pl.BlockSpec((1, _D), lambda i: (0, 0)),
        pl.BlockSpec((_D, _D), lambda i: (0, 0)),
        pl.BlockSpec((1, _D), lambda i: (0, 0)),
        pl.BlockSpec((1, 1, _BR), lambda i: (i, 0, 0)),
        pl.BlockSpec((_D, _MLP_H), lambda i: (0, 0)),
        pl.BlockSpec((1, _MLP_H), lambda i: (0, 0)),
        pl.BlockSpec((_MLP_H, 1), lambda i: (0, 0)),
        pl.BlockSpec((1, 1), lambda i: (0, 0)),
    ],
    out_specs=pl.BlockSpec((_G, 1), lambda i: (0, 0)),
    out_shape=jax.ShapeDtypeStruct((_G, 1), jnp.float32),
    scratch_shapes=[pltpu.VMEM((_G, _D), jnp.float32)],
)


def kernel(graph_feats, edge_index, node2graph,
           W1, b1, R1, rb1, W2, b2, R2, rb2, W3, b3, R3, rb3,
           Wm1, bm1, Wm2, bm2):
    f32 = jnp.float32
    hp = jnp.concatenate(
        [graph_feats, jnp.zeros((_NPAD - _N, _D), f32)], axis=0)
    h = jnp.stack([hp[:, :_H], hp[:, _H:]])
    epad = jnp.full((_EPAD - _E,), _N, jnp.int32)
    srcr = jnp.concatenate([edge_index[0], epad]).reshape(16, _NCHUNK, _CW)
    dstr = jnp.concatenate([edge_index[1], epad]).reshape(16, _NCHUNK, _CW)
    n2gr = jnp.concatenate(
        [node2graph, jnp.full((_NPAD - _N,), _G, jnp.int32)]
    ).reshape(_NPAD // _BR, 1, _BR)
    zrows = jnp.zeros((_SROWS, _H), f32)

    for (W, b, R, rb) in ((W1, b1, R1, rb1), (W2, b2, R2, rb2)):
        agg = _sc_aggregate(h, srcr, dstr, zrows)
        h = _tc_layer(agg, h, W, b.reshape(1, _D), R, rb.reshape(1, _D))
    agg = _sc_aggregate(h, srcr, dstr, zrows)
    return _tc_final(agg, h, W3, b3.reshape(1, _D), R3, rb3.reshape(1, _D),
                     n2gr, Wm1, bm1.reshape(1, _MLP_H), Wm2,
                     bm2.reshape(1, 1))


# speedup vs baseline: 1.2827x; 1.2827x over previous
import functools

import jax
import jax.numpy as jnp
from jax import lax
from jax.experimental import pallas as pl
from jax.experimental.pallas import tpu as pltpu
from jax.experimental.pallas import tpu_sc as plsc

_N, _E, _D, _G = 10000, 320000, 128, 256
_H = _D // 2
_MLP_H = 128
_NPAD = 10240
_CW = 128
_NGRP = 2
_GCH = 81
_NCHUNK = _NGRP * _GCH
_EPAD = 16 * _NCHUNK * _CW
_ROWS_PER_TILE = _NPAD // 16
_SROWS = 128
_NBUF = 3

_MESH = plsc.VectorSubcoreMesh(core_axis_name="c", subcore_axis_name="s")


@functools.partial(
    pl.kernel,
    out_type=jax.ShapeDtypeStruct((2, _NPAD, _H), jnp.float32),
    mesh=_MESH,
    compiler_params=pltpu.CompilerParams(use_tc_tiling_on_sc=False),
    scratch_types=[
        pltpu.VMEM_SHARED((_NPAD, _H), jnp.float32),
        pltpu.VMEM_SHARED((_NPAD, _H), jnp.float32),
        pltpu.VMEM((_GCH, _CW), jnp.int32),
        pltpu.VMEM((_GCH, _CW), jnp.int32),
        pltpu.VMEM((_NBUF, _CW, _H), jnp.float32),
        pltpu.SemaphoreType.DMA((_NBUF,)),
        pltpu.SemaphoreType.DMA((_NBUF,)),
    ],
)
def _sc_aggregate(h_hbm, src_hbm, dst_hbm, zrows_hbm, out_hbm,
                  agg_sh, h_sh, src_v, dst_v, rows_v, sem_g, sem_s):
    c = lax.axis_index("c")
    s = lax.axis_index("s")
    npieces = _ROWS_PER_TILE // _SROWS

    pltpu.sync_copy(zrows_hbm, rows_v.at[0])

    def zstep(k, carry):
        pltpu.sync_copy(
            rows_v.at[0],
            agg_sh.at[pl.ds(s * _ROWS_PER_TILE + k * _SROWS, _SROWS)])
        return carry

    lax.fori_loop(0, npieces, zstep, 0)

    pltpu.async_copy(h_hbm.at[c, pl.ds(s * _ROWS_PER_TILE, _SROWS)],
                     rows_v.at[1], sem_g.at[1])
    for k in range(npieces):
        b = 1 + (k % 2)
        bn = 1 + ((k + 1) % 2)
        base = s * _ROWS_PER_TILE + k * _SROWS
        pltpu.make_async_copy(h_hbm.at[c, pl.ds(base, _SROWS)], rows_v.at[b],
                              sem_g.at[b]).wait()
        if k + 1 < npieces:
            pltpu.async_copy(
                h_hbm.at[c, pl.ds(base + _SROWS, _SROWS)], rows_v.at[bn],
                sem_g.at[bn])
        pltpu.sync_copy(rows_v.at[b], h_sh.at[pl.ds(base, _SROWS)])
    plsc.subcore_barrier()

    for g in range(_NGRP):
        pltpu.sync_copy(src_hbm.at[s, pl.ds(g * _GCH, _GCH)], src_v)
        pltpu.sync_copy(dst_hbm.at[s, pl.ds(g * _GCH, _GCH)], dst_v)

        for b in range(_NBUF):
            pltpu.async_copy(h_sh.at[src_v.at[b]], rows_v.at[b], sem_g.at[b])

        def outer(o, carry):
            for b in range(_NBUF):
                j = o * _NBUF + b
                pltpu.make_async_copy(h_sh.at[src_v.at[j]], rows_v.at[b],
                                      sem_g.at[b]).wait()
                pltpu.async_copy(rows_v.at[b], agg_sh.at[dst_v.at[j]],
                                 sem_s.at[b], add=True)
                bn = (b + 1) % _NBUF

                @pl.when((j >= 2) & (j <= _GCH - 2))
                def _():
                    pltpu.make_async_copy(rows_v.at[bn],
                                          agg_sh.at[dst_v.at[j - 2]],
                                          sem_s.at[bn]).wait()
                    pltpu.async_copy(h_sh.at[src_v.at[j + 1]], rows_v.at[bn],
                                     sem_g.at[bn])
            return carry

        lax.fori_loop(0, _GCH // _NBUF, outer, 0)
        for i in range(_NBUF):
            pltpu.make_async_copy(rows_v.at[i],
                                  agg_sh.at[dst_v.at[_GCH - _NBUF + i]],
                                  sem_s.at[i]).wait()

    plsc.subcore_barrier()

    pltpu.sync_copy(agg_sh.at[pl.ds(s * _ROWS_PER_TILE, _SROWS)], rows_v.at[0])
    for k in range(npieces):
        b = k % 2
        bn = (k + 1) % 2
        base = s * _ROWS_PER_TILE + k * _SROWS
        if k + 1 < npieces:
            pltpu.sync_copy(agg_sh.at[pl.ds(base + _SROWS, _SROWS)],
                            rows_v.at[bn])
        pltpu.sync_copy(rows_v.at[b], out_hbm.at[c, pl.ds(base, _SROWS)])


_BR = 1024


def _tc_layer_body(a_ref, h_ref, w_ref, b_ref, r_ref, rb_ref, o_ref):
    a = jnp.concatenate([a_ref[0], a_ref[1]], axis=1)
    hf = jnp.concatenate([h_ref[0], h_ref[1]], axis=1)
    conv = jnp.dot(a, w_ref[...], preferred_element_type=jnp.float32) + b_ref[...]
    res = jnp.dot(hf, r_ref[...], preferred_element_type=jnp.float32) + rb_ref[...]
    hn = jnp.maximum(conv, 0.0) + jnp.maximum(res, 0.0)
    o_ref[0] = hn[:, :_H]
    o_ref[1] = hn[:, _H:]


_tc_layer = pl.pallas_call(
    _tc_layer_body,
    grid=(_NPAD // _BR,),
    in_specs=[
        pl.BlockSpec((2, _BR, _H), lambda i: (0, i, 0)),
        pl.BlockSpec((2, _BR, _H), lambda i: (0, i, 0)),
        pl.BlockSpec((_D, _D), lambda i: (0, 0)),
        pl.BlockSpec((1, _D), lambda i: (0, 0)),
        pl.BlockSpec((_D, _D), lambda i: (0, 0)),
        pl.BlockSpec((1, _D), lambda i: (0, 0)),
    ],
    out_specs=pl.BlockSpec((2, _BR, _H), lambda i: (0, i, 0)),
    out_shape=jax.ShapeDtypeStruct((2, _NPAD, _H), jnp.float32),
)


def _tc_final_body(a_ref, h_ref, w_ref, b_ref, r_ref, rb_ref, n2g_ref,
                   wm1_ref, bm1_ref, wm2_ref, bm2_ref, o_ref, acc_ref):
    i = pl.program_id(0)
    a = jnp.concatenate([a_ref[0], a_ref[1]], axis=1)
    hf = jnp.concatenate([h_ref[0], h_ref[1]], axis=1)
    conv = jnp.dot(a, w_ref[...], preferred_element_type=jnp.float32) + b_ref[...]
    res = jnp.dot(hf, r_ref[...], preferred_element_type=jnp.float32) + rb_ref[...]
    hn = jnp.maximum(conv, 0.0) + jnp.maximum(res, 0.0)
    gid = lax.broadcasted_iota(jnp.int32, (_G, _BR), 0)
    onehot = (gid == n2g_ref[0]).astype(jnp.float32)
    part = jnp.dot(onehot, hn, preferred_element_type=jnp.float32)

    @pl.when(i == 0)
    def _():
        acc_ref[...] = jnp.zeros_like(acc_ref)

    acc_ref[...] += part

    @pl.when(i == _NPAD // _BR - 1)
    def _():
        mid = jnp.maximum(
            jnp.dot(acc_ref[...], wm1_ref[...],
                    preferred_element_type=jnp.float32) + bm1_ref[...], 0.0)
        o_ref[...] = jnp.dot(mid, wm2_ref[...],
                             preferred_element_type=jnp.float32) + bm2_ref[...]


_tc_final = pl.pallas_call(
    _tc_final_body,
    grid=(_NPAD // _BR,),
    in_specs=[
        pl.BlockSpec((2, _BR, _H), lambda i: (0, i, 0)),
        pl.BlockSpec((2, _BR, _H), lambda i: (0, i, 0)),
        pl.BlockSpec((_D, _D), lambda i: (0, 0)),
        pl.BlockSpec((1, _D), lambda i: (0, 0)),
        pl.BlockSpec((_D, _D), lambda i: (0, 0)),
        pl.BlockSpec((1, _D), lambda i: (0, 0)),
        pl.BlockSpec((1, 1, _BR), lambda i: (i, 0, 0)),
        pl.BlockSpec((_D, _MLP_H), lambda i: (0, 0)),
        pl.BlockSpec((1, _MLP_H), lambda i: (0, 0)),
        pl.BlockSpec((_MLP_H, 1), lambda i: (0, 0)),
        pl.BlockSpec((1, 1), lambda i: (0, 0)),
    ],
    out_specs=pl.BlockSpec((_G, 1), lambda i: (0, 0)),
    out_shape=jax.ShapeDtypeStruct((_G, 1), jnp.float32),
    scratch_shapes=[pltpu.VMEM((_G, _D), jnp.float32)],
)


def kernel(graph_feats, edge_index, node2graph,
           W1, b1, R1, rb1, W2, b2, R2, rb2, W3, b3, R3, rb3,
           Wm1, bm1, Wm2, bm2):
    f32 = jnp.float32
    hp = jnp.concatenate(
        [graph_feats, jnp.zeros((_NPAD - _N, _D), f32)], axis=0)
    h = jnp.stack([hp[:, :_H], hp[:, _H:]])
    epad = jnp.full((_EPAD - _E,), _N, jnp.int32)
    srcr = jnp.concatenate([edge_index[0], epad]).reshape(16, _NCHUNK, _CW)
    dstr = jnp.concatenate([edge_index[1], epad]).reshape(16, _NCHUNK, _CW)
    n2gr = jnp.concatenate(
        [node2graph, jnp.full((_NPAD - _N,), _G, jnp.int32)]
    ).reshape(_NPAD // _BR, 1, _BR)
    zrows = jnp.zeros((_SROWS, _H), f32)

    for (W, b, R, rb) in ((W1, b1, R1, rb1), (W2, b2, R2, rb2)):
        agg = _sc_aggregate(h, srcr, dstr, zrows)
        h = _tc_layer(agg, h, W, b.reshape(1, _D), R, rb.reshape(1, _D))
    agg = _sc_aggregate(h, srcr, dstr, zrows)
    return _tc_final(agg, h, W3, b3.reshape(1, _D), R3, rb3.reshape(1, _D),
                     n2gr, Wm1, bm1.reshape(1, _MLP_H), Wm2,
                     bm2.reshape(1, 1))


# speedup vs baseline: 1.2959x; 1.0103x over previous
import functools

import jax
import jax.numpy as jnp
from jax import lax
from jax.experimental import pallas as pl
from jax.experimental.pallas import tpu as pltpu
from jax.experimental.pallas import tpu_sc as plsc

_N, _E, _D, _G = 10000, 320000, 128, 256
_H = _D // 2
_MLP_H = 128
_NPAD = 10240
_CW = 128
_NGRP = 3
_GCH = 56
_NCHUNK = _NGRP * _GCH
_EPAD = 16 * _NCHUNK * _CW
_ROWS_PER_TILE = _NPAD // 16
_SROWS = 128
_NBUF = 4

_MESH = plsc.VectorSubcoreMesh(core_axis_name="c", subcore_axis_name="s")


@functools.partial(
    pl.kernel,
    out_type=jax.ShapeDtypeStruct((2, _NPAD, _H), jnp.float32),
    mesh=_MESH,
    compiler_params=pltpu.CompilerParams(use_tc_tiling_on_sc=False),
    scratch_types=[
        pltpu.VMEM_SHARED((_NPAD, _H), jnp.float32),
        pltpu.VMEM_SHARED((_NPAD, _H), jnp.float32),
        pltpu.VMEM((_GCH, _CW), jnp.int32),
        pltpu.VMEM((_GCH, _CW), jnp.int32),
        pltpu.VMEM((_NBUF, _CW, _H), jnp.float32),
        pltpu.SemaphoreType.DMA((_NBUF,)),
        pltpu.SemaphoreType.DMA((_NBUF,)),
    ],
)
def _sc_aggregate(h_hbm, src_hbm, dst_hbm, zrows_hbm, out_hbm,
                  agg_sh, h_sh, src_v, dst_v, rows_v, sem_g, sem_s):
    c = lax.axis_index("c")
    s = lax.axis_index("s")
    npieces = _ROWS_PER_TILE // _SROWS

    pltpu.sync_copy(zrows_hbm, rows_v.at[0])

    def zstep(k, carry):
        pltpu.sync_copy(
            rows_v.at[0],
            agg_sh.at[pl.ds(s * _ROWS_PER_TILE + k * _SROWS, _SROWS)])
        return carry

    lax.fori_loop(0, npieces, zstep, 0)

    pltpu.async_copy(h_hbm.at[c, pl.ds(s * _ROWS_PER_TILE, _SROWS)],
                     rows_v.at[1], sem_g.at[1])
    for k in range(npieces):
        b = 1 + (k % 2)
        bn = 1 + ((k + 1) % 2)
        base = s * _ROWS_PER_TILE + k * _SROWS
        pltpu.make_async_copy(h_hbm.at[c, pl.ds(base, _SROWS)], rows_v.at[b],
                              sem_g.at[b]).wait()
        if k + 1 < npieces:
            pltpu.async_copy(
                h_hbm.at[c, pl.ds(base + _SROWS, _SROWS)], rows_v.at[bn],
                sem_g.at[bn])
        pltpu.sync_copy(rows_v.at[b], h_sh.at[pl.ds(base, _SROWS)])
    plsc.subcore_barrier()

    for g in range(_NGRP):
        pltpu.sync_copy(src_hbm.at[s, pl.ds(g * _GCH, _GCH)], src_v)
        pltpu.sync_copy(dst_hbm.at[s, pl.ds(g * _GCH, _GCH)], dst_v)

        for b in range(_NBUF):
            pltpu.async_copy(h_sh.at[src_v.at[b]], rows_v.at[b], sem_g.at[b])

        def outer(o, carry):
            for b in range(_NBUF):
                j = o * _NBUF + b
                pltpu.make_async_copy(h_sh.at[src_v.at[j]], rows_v.at[b],
                                      sem_g.at[b]).wait()
                pltpu.async_copy(rows_v.at[b], agg_sh.at[dst_v.at[j]],
                                 sem_s.at[b], add=True)
                bn = (b + 2) % _NBUF

                @pl.when((j >= 2) & (j <= _GCH - 3))
                def _():
                    pltpu.make_async_copy(rows_v.at[bn],
                                          agg_sh.at[dst_v.at[j - 2]],
                                          sem_s.at[bn]).wait()
                    pltpu.async_copy(h_sh.at[src_v.at[j + 2]], rows_v.at[bn],
                                     sem_g.at[bn])
            return carry

        lax.fori_loop(0, _GCH // _NBUF, outer, 0)
        for i in range(_NBUF):
            pltpu.make_async_copy(rows_v.at[i],
                                  agg_sh.at[dst_v.at[_GCH - _NBUF + i]],
                                  sem_s.at[i]).wait()

    plsc.subcore_barrier()

    pltpu.sync_copy(agg_sh.at[pl.ds(s * _ROWS_PER_TILE, _SROWS)], rows_v.at[0])
    for k in range(npieces):
        b = k % 2
        bn = (k + 1) % 2
        base = s * _ROWS_PER_TILE + k * _SROWS
        if k + 1 < npieces:
            pltpu.sync_copy(agg_sh.at[pl.ds(base + _SROWS, _SROWS)],
                            rows_v.at[bn])
        pltpu.sync_copy(rows_v.at[b], out_hbm.at[c, pl.ds(base, _SROWS)])


_BR = 1024


def _tc_layer_body(a_ref, h_ref, w_ref, b_ref, r_ref, rb_ref, o_ref):
    a = jnp.concatenate([a_ref[0], a_ref[1]], axis=1)
    hf = jnp.concatenate([h_ref[0], h_ref[1]], axis=1)
    conv = jnp.dot(a, w_ref[...], preferred_element_type=jnp.float32) + b_ref[...]
    res = jnp.dot(hf, r_ref[...], preferred_element_type=jnp.float32) + rb_ref[...]
    hn = jnp.maximum(conv, 0.0) + jnp.maximum(res, 0.0)
    o_ref[0] = hn[:, :_H]
    o_ref[1] = hn[:, _H:]


_tc_layer = pl.pallas_call(
    _tc_layer_body,
    grid=(_NPAD // _BR,),
    in_specs=[
        pl.BlockSpec((2, _BR, _H), lambda i: (0, i, 0)),
        pl.BlockSpec((2, _BR, _H), lambda i: (0, i, 0)),
        pl.BlockSpec((_D, _D), lambda i: (0, 0)),
        pl.BlockSpec((1, _D), lambda i: (0, 0)),
        pl.BlockSpec((_D, _D), lambda i: (0, 0)),
        pl.BlockSpec((1, _D), lambda i: (0, 0)),
    ],
    out_specs=pl.BlockSpec((2, _BR, _H), lambda i: (0, i, 0)),
    out_shape=jax.ShapeDtypeStruct((2, _NPAD, _H), jnp.float32),
)


def _tc_final_body(a_ref, h_ref, w_ref, b_ref, r_ref, rb_ref, n2g_ref,
                   wm1_ref, bm1_ref, wm2_ref, bm2_ref, o_ref, acc_ref):
    i = pl.program_id(0)
    a = jnp.concatenate([a_ref[0], a_ref[1]], axis=1)
    hf = jnp.concatenate([h_ref[0], h_ref[1]], axis=1)
    conv = jnp.dot(a, w_ref[...], preferred_element_type=jnp.float32) + b_ref[...]
    res = jnp.dot(hf, r_ref[...], preferred_element_type=jnp.float32) + rb_ref[...]
    hn = jnp.maximum(conv, 0.0) + jnp.maximum(res, 0.0)
    gid = lax.broadcasted_iota(jnp.int32, (_G, _BR), 0)
    onehot = (gid == n2g_ref[0]).astype(jnp.float32)
    part = jnp.dot(onehot, hn, preferred_element_type=jnp.float32)

    @pl.when(i == 0)
    def _():
        acc_ref[...] = jnp.zeros_like(acc_ref)

    acc_ref[...] += part

    @pl.when(i == _NPAD // _BR - 1)
    def _():
        mid = jnp.maximum(
            jnp.dot(acc_ref[...], wm1_ref[...],
                    preferred_element_type=jnp.float32) + bm1_ref[...], 0.0)
        o_ref[...] = jnp.dot(mid, wm2_ref[...],
                             preferred_element_type=jnp.float32) + bm2_ref[...]


_tc_final = pl.pallas_call(
    _tc_final_body,
    grid=(_NPAD // _BR,),
    in_specs=[
        pl.BlockSpec((2, _BR, _H), lambda i: (0, i, 0)),
        pl.BlockSpec((2, _BR, _H), lambda i: (0, i, 0)),
        pl.BlockSpec((_D, _D), lambda i: (0, 0)),
        pl.BlockSpec((1, _D), lambda i: (0, 0)),
        pl.BlockSpec((_D, _D), lambda i: (0, 0)),
        pl.BlockSpec((1, _D), lambda i: (0, 0)),
        pl.BlockSpec((1, 1, _BR), lambda i: (i, 0, 0)),
        pl.BlockSpec((_D, _MLP_H), lambda i: (0, 0)),
        pl.BlockSpec((1, _MLP_H), lambda i: (0, 0)),
        pl.BlockSpec((_MLP_H, 1), lambda i: (0, 0)),
        pl.BlockSpec((1, 1), lambda i: (0, 0)),
    ],
    out_specs=pl.BlockSpec((_G, 1), lambda i: (0, 0)),
    out_shape=jax.ShapeDtypeStruct((_G, 1), jnp.float32),
    scratch_shapes=[pltpu.VMEM((_G, _D), jnp.float32)],
)


def kernel(graph_feats, edge_index, node2graph,
           W1, b1, R1, rb1, W2, b2, R2, rb2, W3, b3, R3, rb3,
           Wm1, bm1, Wm2, bm2):
    f32 = jnp.float32
    hp = jnp.concatenate(
        [graph_feats, jnp.zeros((_NPAD - _N, _D), f32)], axis=0)
    h = jnp.stack([hp[:, :_H], hp[:, _H:]])
    epad = jnp.full((_EPAD - _E,), _N, jnp.int32)
    srcr = jnp.concatenate([edge_index[0], epad]).reshape(16, _NCHUNK, _CW)
    dstr = jnp.concatenate([edge_index[1], epad]).reshape(16, _NCHUNK, _CW)
    n2gr = jnp.concatenate(
        [node2graph, jnp.full((_NPAD - _N,), _G, jnp.int32)]
    ).reshape(_NPAD // _BR, 1, _BR)
    zrows = jnp.zeros((_SROWS, _H), f32)

    for (W, b, R, rb) in ((W1, b1, R1, rb1), (W2, b2, R2, rb2)):
        agg = _sc_aggregate(h, srcr, dstr, zrows)
        h = _tc_layer(agg, h, W, b.reshape(1, _D), R, rb.reshape(1, _D))
    agg = _sc_aggregate(h, srcr, dstr, zrows)
    return _tc_final(agg, h, W3, b3.reshape(1, _D), R3, rb3.reshape(1, _D),
                     n2gr, Wm1, bm1.reshape(1, _MLP_H), Wm2,
                     bm2.reshape(1, 1))
